# baseline (speedup 1.0000x reference)
import functools

import jax
import jax.numpy as jnp
from jax.experimental import pallas as pl
from jax.experimental.pallas import tpu as pltpu

B, N, D = 1, 2048, 1024
E, K, H = 8, 2, 256
GATE_SCALE = 0.5
TT = 1024
NT = N // TT


def _moe_kernel(x_ref, wg_ref, w1_ref, b1_ref, w2_ref, b2_ref,
                y_ref, loss_ref, imp_ref, load_ref, abs_ref,
                w1b_ref, w2b_ref):
    i = pl.program_id(0)

    xt = x_ref[...]
    logits = jnp.dot(xt, wg_ref[...], preferred_element_type=jnp.float32)

    eidx = jax.lax.broadcasted_iota(jnp.int32, (TT, E), 1)
    m1 = jnp.max(logits, axis=1, keepdims=True)
    i1 = jnp.min(jnp.where(logits == m1, eidx, E), axis=1, keepdims=True)
    sel1 = (eidx == i1)
    l2 = jnp.where(sel1, -jnp.inf, logits)
    m2 = jnp.max(l2, axis=1, keepdims=True)
    i2 = jnp.min(jnp.where(l2 == m2, eidx, E), axis=1, keepdims=True)
    sel2 = (eidx == i2)

    e2 = jnp.exp(m2 - m1)
    denom = 1.0 + e2
    g1 = 1.0 / denom
    g2 = e2 / denom
    gates = jnp.where(sel1, g1, 0.0) + jnp.where(sel2, g2, 0.0)

    @pl.when(i == 0)
    def _():
        imp_ref[...] = jnp.zeros_like(imp_ref)
        load_ref[...] = jnp.zeros_like(load_ref)
        abs_ref[...] = jnp.zeros_like(abs_ref)
        w1b_ref[...] = jnp.swapaxes(w1_ref[...].astype(jnp.bfloat16), 0, 1)
        w2b_ref[...] = jnp.swapaxes(
            (w2_ref[...] * GATE_SCALE).astype(jnp.bfloat16), 1, 2)

    imp_ref[...] += jnp.sum(gates, axis=0, keepdims=True)
    load_ref[...] += jnp.sum((gates > 0).astype(jnp.float32), axis=0,
                             keepdims=True)

    xb = xt.astype(jnp.bfloat16)
    h = jnp.dot(xb, w1b_ref[...], preferred_element_type=jnp.float32)
    h = jnp.maximum(h + b1_ref[...], 0.0)
    b2s = (b2_ref[...] * GATE_SCALE).astype(jnp.bfloat16)
    y = jnp.dot(gates.astype(jnp.bfloat16), b2s,
                preferred_element_type=jnp.float32)
    for e in range(E):
        hg = (h[:, e * H:(e + 1) * H] *
              gates[:, e:e + 1]).astype(jnp.bfloat16)
        y = y + jnp.dot(hg, w2b_ref[e],
                        preferred_element_type=jnp.float32)

    abs_ref[...] += jnp.sum(jnp.abs(y), axis=0, keepdims=True)
    y_ref[...] = y

    @pl.when(i == NT - 1)
    def _():
        eps = jnp.float32(1e-10)
        inv_e = jnp.float32(1.0 / E)
        inv_em1 = jnp.float32(1.0 / (E - 1))

        def cv_sq(v):
            mean = jnp.sum(v) * inv_e
            var = jnp.sum((v - mean) ** 2) * inv_em1
            return var / (mean * mean + eps)

        loss = cv_sq(imp_ref[...]) + cv_sq(load_ref[...])
        loss = loss + jnp.sum(abs_ref[...]) * jnp.float32(
            (1.0 / GATE_SCALE) / (N * D))
        loss_ref[...] = jnp.broadcast_to(loss, (1, 1))


@functools.partial(jax.jit, static_argnames=())
def _run(xf, w_gate, w1, b1, w2, b2):
    y, loss = pl.pallas_call(
        _moe_kernel,
        grid=(NT,),
        in_specs=[
            pl.BlockSpec((TT, D), lambda i: (i, 0)),
            pl.BlockSpec((D, E), lambda i: (0, 0)),
            pl.BlockSpec((E * H, D), lambda i: (0, 0)),
            pl.BlockSpec((1, E * H), lambda i: (0, 0)),
            pl.BlockSpec((E, D, H), lambda i: (0, 0, 0)),
            pl.BlockSpec((E, D), lambda i: (0, 0)),
        ],
        out_specs=[
            pl.BlockSpec((TT, D), lambda i: (i, 0)),
            pl.BlockSpec((1, 1), lambda i: (0, 0)),
        ],
        out_shape=[
            jax.ShapeDtypeStruct((N, D), jnp.float32),
            jax.ShapeDtypeStruct((1, 1), jnp.float32),
        ],
        scratch_shapes=[
            pltpu.VMEM((1, E), jnp.float32),
            pltpu.VMEM((1, E), jnp.float32),
            pltpu.VMEM((1, D), jnp.float32),
            pltpu.VMEM((D, E * H), jnp.bfloat16),
            pltpu.VMEM((E, H, D), jnp.bfloat16),
        ],
        compiler_params=pltpu.CompilerParams(
            dimension_semantics=("arbitrary",),
        ),
    )(xf, w_gate, w1, b1, w2, b2)
    return y, loss


def kernel(x, w_gate, fc1_w, fc1_b, fc2_w, fc2_b):
    b, n, d = x.shape
    xf = x.reshape(n, d)
    w1 = fc1_w.reshape(E * H, d)
    b1 = fc1_b.reshape(1, E * H)
    y, loss = _run(xf, w_gate, w1, b1, fc2_w, fc2_b)
    return y.reshape(b, n, d), loss[0, 0]


# speedup vs baseline: 1.0394x; 1.0394x over previous
import functools

import jax
import jax.numpy as jnp
from jax.experimental import pallas as pl
from jax.experimental.pallas import tpu as pltpu

B, N, D = 1, 2048, 1024
E, K, H = 8, 2, 256
GATE_SCALE = 0.5
TT = 1024
NT = N // TT


def _moe_kernel(x_ref, wg_ref, w1_ref, b1_ref, w2_ref, b2_ref,
                y_ref, loss_ref, imp_ref, load_ref, abs_ref,
                w1b_ref, w2b_ref):
    i = pl.program_id(0)

    xt = x_ref[...]
    logits = jnp.dot(xt, wg_ref[...], preferred_element_type=jnp.float32)

    eidx = jax.lax.broadcasted_iota(jnp.int32, (TT, E), 1)
    m1 = jnp.max(logits, axis=1, keepdims=True)
    i1 = jnp.min(jnp.where(logits == m1, eidx, E), axis=1, keepdims=True)
    sel1 = (eidx == i1)
    l2 = jnp.where(sel1, -jnp.inf, logits)
    m2 = jnp.max(l2, axis=1, keepdims=True)
    i2 = jnp.min(jnp.where(l2 == m2, eidx, E), axis=1, keepdims=True)
    sel2 = (eidx == i2)

    e2 = jnp.exp(m2 - m1)
    denom = 1.0 + e2
    g1 = 1.0 / denom
    g2 = e2 / denom
    gates = jnp.where(sel1, g1, 0.0) + jnp.where(sel2, g2, 0.0)

    @pl.when(i == 0)
    def _():
        imp_ref[...] = jnp.zeros_like(imp_ref)
        load_ref[...] = jnp.zeros_like(load_ref)
        abs_ref[...] = jnp.zeros_like(abs_ref)
        w1b_ref[...] = w1_ref[...].astype(jnp.bfloat16)
        w2b_ref[...] = (w2_ref[...] * GATE_SCALE).astype(jnp.bfloat16)

    imp_ref[...] += jnp.sum(gates, axis=0, keepdims=True)
    load_ref[...] += jnp.sum((gates > 0).astype(jnp.float32), axis=0,
                             keepdims=True)

    xb = xt.astype(jnp.bfloat16)
    h = jax.lax.dot_general(xb, w1b_ref[...], (((1,), (1,)), ((), ())),
                            preferred_element_type=jnp.float32)
    h = jnp.maximum(h + b1_ref[...], 0.0)
    b2s = (b2_ref[...] * GATE_SCALE).astype(jnp.bfloat16)
    y = jnp.dot(gates.astype(jnp.bfloat16), b2s,
                preferred_element_type=jnp.float32)
    for e in range(E):
        hg = (h[:, e * H:(e + 1) * H] *
              gates[:, e:e + 1]).astype(jnp.bfloat16)
        y = y + jax.lax.dot_general(
            hg, w2b_ref[e], (((1,), (1,)), ((), ())),
            preferred_element_type=jnp.float32)

    abs_ref[...] += jnp.sum(jnp.abs(y), axis=0, keepdims=True)
    y_ref[...] = y

    @pl.when(i == NT - 1)
    def _():
        eps = jnp.float32(1e-10)
        inv_e = jnp.float32(1.0 / E)
        inv_em1 = jnp.float32(1.0 / (E - 1))

        def cv_sq(v):
            mean = jnp.sum(v) * inv_e
            var = jnp.sum((v - mean) ** 2) * inv_em1
            return var / (mean * mean + eps)

        loss = cv_sq(imp_ref[...]) + cv_sq(load_ref[...])
        loss = loss + jnp.sum(abs_ref[...]) * jnp.float32(
            (1.0 / GATE_SCALE) / (N * D))
        loss_ref[...] = jnp.broadcast_to(loss, (1, 1))


@functools.partial(jax.jit, static_argnames=())
def _run(xf, w_gate, w1, b1, w2, b2):
    y, loss = pl.pallas_call(
        _moe_kernel,
        grid=(NT,),
        in_specs=[
            pl.BlockSpec((TT, D), lambda i: (i, 0)),
            pl.BlockSpec((D, E), lambda i: (0, 0)),
            pl.BlockSpec((E * H, D), lambda i: (0, 0)),
            pl.BlockSpec((1, E * H), lambda i: (0, 0)),
            pl.BlockSpec((E, D, H), lambda i: (0, 0, 0)),
            pl.BlockSpec((E, D), lambda i: (0, 0)),
        ],
        out_specs=[
            pl.BlockSpec((TT, D), lambda i: (i, 0)),
            pl.BlockSpec((1, 1), lambda i: (0, 0)),
        ],
        out_shape=[
            jax.ShapeDtypeStruct((N, D), jnp.float32),
            jax.ShapeDtypeStruct((1, 1), jnp.float32),
        ],
        scratch_shapes=[
            pltpu.VMEM((1, E), jnp.float32),
            pltpu.VMEM((1, E), jnp.float32),
            pltpu.VMEM((1, D), jnp.float32),
            pltpu.VMEM((E * H, D), jnp.bfloat16),
            pltpu.VMEM((E, D, H), jnp.bfloat16),
        ],
        compiler_params=pltpu.CompilerParams(
            dimension_semantics=("arbitrary",),
        ),
    )(xf, w_gate, w1, b1, w2, b2)
    return y, loss


def kernel(x, w_gate, fc1_w, fc1_b, fc2_w, fc2_b):
    b, n, d = x.shape
    xf = x.reshape(n, d)
    w1 = fc1_w.reshape(E * H, d)
    b1 = fc1_b.reshape(1, E * H)
    y, loss = _run(xf, w_gate, w1, b1, fc2_w, fc2_b)
    return y.reshape(b, n, d), loss[0, 0]


# speedup vs baseline: 1.1278x; 1.0851x over previous
import functools

import jax
import jax.numpy as jnp
from jax.experimental import pallas as pl
from jax.experimental.pallas import tpu as pltpu

B, N, D = 1, 2048, 1024
E, K, H = 8, 2, 256
GATE_SCALE = 0.5
TT = 1024
NT = N // TT


def _moe_kernel(x_ref, wg_ref, w1_ref, b1_ref, w2_ref, b2_ref,
                y_ref, loss_ref, imp_ref, load_ref, abs_ref,
                w1b_ref, w2b_ref):
    i = pl.program_id(0)

    xt = x_ref[...]
    logits = jnp.dot(xt, wg_ref[...], preferred_element_type=jnp.float32)

    eidx = jax.lax.broadcasted_iota(jnp.int32, (TT, E), 1)
    m1 = jnp.max(logits, axis=1, keepdims=True)
    i1 = jnp.min(jnp.where(logits == m1, eidx, E), axis=1, keepdims=True)
    sel1 = (eidx == i1)
    l2 = jnp.where(sel1, -jnp.inf, logits)
    m2 = jnp.max(l2, axis=1, keepdims=True)
    i2 = jnp.min(jnp.where(l2 == m2, eidx, E), axis=1, keepdims=True)
    sel2 = (eidx == i2)

    e2 = jnp.exp(m2 - m1)
    denom = 1.0 + e2
    g1 = 1.0 / denom
    g2 = e2 / denom
    gates = jnp.where(sel1, g1, 0.0) + jnp.where(sel2, g2, 0.0)

    @pl.when(i == 0)
    def _():
        imp_ref[...] = jnp.zeros_like(imp_ref)
        load_ref[...] = jnp.zeros_like(load_ref)
        abs_ref[...] = jnp.zeros_like(abs_ref)
        w1b_ref[...] = w1_ref[...].astype(jnp.bfloat16)
        w2b_ref[...] = jnp.swapaxes(
            (w2_ref[...] * GATE_SCALE).astype(jnp.bfloat16),
            1, 2).reshape(E * H, D)

    imp_ref[...] += jnp.sum(gates, axis=0, keepdims=True)
    load_ref[...] += jnp.sum((gates > 0).astype(jnp.float32), axis=0,
                             keepdims=True)

    xb = xt.astype(jnp.bfloat16)
    h = jax.lax.dot_general(xb, w1b_ref[...], (((1,), (1,)), ((), ())),
                            preferred_element_type=jnp.float32)
    h = jnp.maximum(h + b1_ref[...], 0.0)
    b2s = (b2_ref[...] * GATE_SCALE).astype(jnp.bfloat16)
    y = jnp.dot(gates.astype(jnp.bfloat16), b2s,
                preferred_element_type=jnp.float32)
    hg = jnp.concatenate(
        [(h[:, e * H:(e + 1) * H] * gates[:, e:e + 1]).astype(jnp.bfloat16)
         for e in range(E)], axis=1)
    y = y + jnp.dot(hg, w2b_ref[...], preferred_element_type=jnp.float32)

    abs_ref[...] += jnp.sum(jnp.abs(y), axis=0, keepdims=True)
    y_ref[...] = y

    @pl.when(i == NT - 1)
    def _():
        eps = jnp.float32(1e-10)
        inv_e = jnp.float32(1.0 / E)
        inv_em1 = jnp.float32(1.0 / (E - 1))

        def cv_sq(v):
            mean = jnp.sum(v) * inv_e
            var = jnp.sum((v - mean) ** 2) * inv_em1
            return var / (mean * mean + eps)

        loss = cv_sq(imp_ref[...]) + cv_sq(load_ref[...])
        loss = loss + jnp.sum(abs_ref[...]) * jnp.float32(
            (1.0 / GATE_SCALE) / (N * D))
        loss_ref[...] = jnp.broadcast_to(loss, (1, 1))


@functools.partial(jax.jit, static_argnames=())
def _run(xf, w_gate, w1, b1, w2, b2):
    y, loss = pl.pallas_call(
        _moe_kernel,
        grid=(NT,),
        in_specs=[
            pl.BlockSpec((TT, D), lambda i: (i, 0)),
            pl.BlockSpec((D, E), lambda i: (0, 0)),
            pl.BlockSpec((E * H, D), lambda i: (0, 0)),
            pl.BlockSpec((1, E * H), lambda i: (0, 0)),
            pl.BlockSpec((E, D, H), lambda i: (0, 0, 0)),
            pl.BlockSpec((E, D), lambda i: (0, 0)),
        ],
        out_specs=[
            pl.BlockSpec((TT, D), lambda i: (i, 0)),
            pl.BlockSpec((1, 1), lambda i: (0, 0)),
        ],
        out_shape=[
            jax.ShapeDtypeStruct((N, D), jnp.float32),
            jax.ShapeDtypeStruct((1, 1), jnp.float32),
        ],
        scratch_shapes=[
            pltpu.VMEM((1, E), jnp.float32),
            pltpu.VMEM((1, E), jnp.float32),
            pltpu.VMEM((1, D), jnp.float32),
            pltpu.VMEM((E * H, D), jnp.bfloat16),
            pltpu.VMEM((E * H, D), jnp.bfloat16),
        ],
        compiler_params=pltpu.CompilerParams(
            dimension_semantics=("arbitrary",),
        ),
    )(xf, w_gate, w1, b1, w2, b2)
    return y, loss


def kernel(x, w_gate, fc1_w, fc1_b, fc2_w, fc2_b):
    b, n, d = x.shape
    xf = x.reshape(n, d)
    w1 = fc1_w.reshape(E * H, d)
    b1 = fc1_b.reshape(1, E * H)
    y, loss = _run(xf, w_gate, w1, b1, fc2_w, fc2_b)
    return y.reshape(b, n, d), loss[0, 0]
